# Initial kernel scaffold; baseline (speedup 1.0000x reference)
import jax
import jax.numpy as jnp
from jax.experimental import pallas as pl


def kernel(student_output):
    raise NotImplementedError("write your pallas kernel here")


# speedup vs baseline: 2.8457x; 2.8457x over previous
import jax
import jax.numpy as jnp
from jax.experimental import pallas as pl
from jax.experimental.pallas import tpu as pltpu

_N = 4096
_D = 1024
_T = 512
_NT = _N // _T
_EPS = 1e-8


def _koleo_kernel(x_ref, out_ref, xb_ref, sqr_ref, sqc_ref, rowacc_ref,
                  colacc_ref):
    x = x_ref[:]
    xb_ref[:] = x.astype(jnp.bfloat16)
    sq = jnp.sum(x * x, axis=1, keepdims=True)
    sqr_ref[:] = sq
    sqc_ref[:] = sq.reshape(1, _N)
    rowacc_ref[:] = jnp.full((_N, 1), jnp.inf, jnp.float32)
    colacc_ref[:] = jnp.full((1, _N), jnp.inf, jnp.float32)

    for i in range(_NT):
        xi = xb_ref[i * _T:(i + 1) * _T, :]
        for j in range(i, _NT):
            xj = xb_ref[j * _T:(j + 1) * _T, :]
            g = jax.lax.dot_general(
                xi, xj, (((1,), (1,)), ((), ())),
                preferred_element_type=jnp.float32)
            if i == j:
                rr = jax.lax.broadcasted_iota(jnp.int32, (_T, _T), 0)
                cc = jax.lax.broadcasted_iota(jnp.int32, (_T, _T), 1)
                g = jnp.where(rr == cc, -jnp.inf, g)
            sqj = sqc_ref[:, j * _T:(j + 1) * _T]
            t1 = sqj - 2.0 * g
            rowacc_ref[i * _T:(i + 1) * _T, :] = jnp.minimum(
                rowacc_ref[i * _T:(i + 1) * _T, :],
                jnp.min(t1, axis=1, keepdims=True))
            if i != j:
                sqi = sqr_ref[i * _T:(i + 1) * _T, :]
                t2 = sqi - 2.0 * g
                colacc_ref[:, j * _T:(j + 1) * _T] = jnp.minimum(
                    colacc_ref[:, j * _T:(j + 1) * _T],
                    jnp.min(t2, axis=0, keepdims=True))

    rowfull = rowacc_ref[:] + sqr_ref[:]
    colfull = (colacc_ref[:] + sqc_ref[:]).reshape(_N, 1)
    md2 = jnp.maximum(jnp.minimum(rowfull, colfull), 0.0)
    s = jnp.sum(jnp.log(jnp.sqrt(md2) + _EPS), keepdims=True)
    out_ref[:, :] = s * (-1.0 / _N)


def kernel(student_output):
    out = pl.pallas_call(
        _koleo_kernel,
        out_shape=jax.ShapeDtypeStruct((1, 1), jnp.float32),
        scratch_shapes=[
            pltpu.VMEM((_N, _D), jnp.bfloat16),
            pltpu.VMEM((_N, 1), jnp.float32),
            pltpu.VMEM((1, _N), jnp.float32),
            pltpu.VMEM((_N, 1), jnp.float32),
            pltpu.VMEM((1, _N), jnp.float32),
        ],
    )(student_output)
    return out[0, 0]
